# baseline (speedup 1.0000x reference)
import functools

import jax
import jax.numpy as jnp
from jax import lax
from jax.experimental import pallas as pl
from jax.experimental.pallas import tpu as pltpu
from jax.experimental.pallas import tpu_sc as plsc

B = 16384
L = 12
V = 29
H = 256
N = B * L
TROWS = L * V

N_SC = 36864
N_TC = N - N_SC
NW = 32
ROWS_PER_W = N_SC // NW
CROWS = 32
NCH = ROWS_PER_W // CROWS
LPAT = 384
TCR = 6144
TC_BLOCKS = N_TC // TCR


def _sc_body(x_hbm, word_hbm, pos_hbm, out_hbm,
             xv, lpat, wordv, posv, tab, bufa, bufb, sema, semb):
    c = lax.axis_index("c")
    s = lax.axis_index("s")
    wid = s * 2 + c
    base = wid * ROWS_PER_W
    iota = lax.iota(jnp.int32, 16)

    pltpu.sync_copy(x_hbm.at[pl.ds(base, ROWS_PER_W)], xv)
    pltpu.sync_copy(word_hbm, wordv)
    pltpu.sync_copy(pos_hbm, posv)

    def build_l(l, carry):
        pv = [posv[l, pl.ds(16 * j, 16)] for j in range(H // 16)]
        rb = (l * V) * H
        for v in range(V):
            for j in range(H // 16):
                tab[pl.ds(rb + v * H + 16 * j, 16)] = (
                    wordv[v, pl.ds(16 * j, 16)] + pv[j]
                )
        return carry

    lax.fori_loop(0, L, build_l, 0)

    for i in range(LPAT // 16):
        r = iota + jnp.int32(16 * i)
        lpat[pl.ds(16 * i, 16)] = lax.rem(r, jnp.int32(L)) * jnp.int32(V)

    def cstep(i, carry):
        ph = lax.rem(i, jnp.int32(LPAT // 16)) * 16
        xv[pl.ds(i * 16, 16)] = xv[pl.ds(i * 16, 16)] + lpat[pl.ds(ph, 16)]
        return carry

    lax.fori_loop(0, ROWS_PER_W // 16, cstep, 0)

    def fill(k, buf):
        def group(g, carry):
            rv = xv[pl.ds(k * CROWS + g * 16, 16)] * jnp.int32(H)
            for j in range(16):
                src = rv[j]
                dst = (g * 16 + j) * H
                vals = [tab[pl.ds(src + 16 * ci, 16)] for ci in range(H // 16)]
                for ci in range(H // 16):
                    buf[pl.ds(dst + 16 * ci, 16)] = vals[ci]
            return carry

        lax.fori_loop(0, CROWS // 16, group, 0)

    def start_write(k, buf, sem):
        pltpu.async_copy(
            buf, out_hbm.at[pl.ds((base + k * CROWS) * H, CROWS * H)], sem
        )

    def wait_write(buf, sem):
        pltpu.make_async_copy(
            buf, out_hbm.at[pl.ds(base * H, CROWS * H)], sem
        ).wait()

    fill(0, bufa)
    start_write(0, bufa, sema)
    fill(1, bufb)
    start_write(1, bufb, semb)

    def step(i, carry):
        k0 = 2 * i
        wait_write(bufa, sema)
        fill(k0, bufa)
        start_write(k0, bufa, sema)
        wait_write(bufb, semb)
        fill(k0 + 1, bufb)
        start_write(k0 + 1, bufb, semb)
        return carry

    lax.fori_loop(1, NCH // 2, step, 0)
    wait_write(bufa, sema)
    wait_write(bufb, semb)


def _tc_body(x_ref, word_ref, pos_ref, o_ref):
    tabf = (pos_ref[...][:, None, :] + word_ref[...][None, :, :]
            ).reshape(TROWS, H)
    rloc = lax.broadcasted_iota(jnp.int32, (TCR, 1), 0)
    idx = x_ref[...].reshape(TCR, 1) + lax.rem(rloc, jnp.int32(L)) * jnp.int32(V)
    onehot = (idx == lax.broadcasted_iota(jnp.int32, (TCR, TROWS), 1)
              ).astype(jnp.float32)
    o_ref[...] = jnp.dot(onehot, tabf, preferred_element_type=jnp.float32,
                         precision=lax.Precision.HIGHEST)


@jax.jit
def _ebd(x_flat, word_ebd, pos_ebd):
    mesh = plsc.VectorSubcoreMesh(core_axis_name="c", subcore_axis_name="s")
    sc = functools.partial(
        pl.kernel,
        mesh=mesh,
        compiler_params=pltpu.CompilerParams(
            needs_layout_passes=False,
            use_tc_tiling_on_sc=False,
        ),
        out_type=jax.ShapeDtypeStruct((N_SC * H,), jnp.float32),
        scratch_types=[
            pltpu.VMEM((ROWS_PER_W,), jnp.int32),
            pltpu.VMEM((LPAT,), jnp.int32),
            pltpu.VMEM((V, H), jnp.float32),
            pltpu.VMEM((L, H), jnp.float32),
            pltpu.VMEM((TROWS * H,), jnp.float32),
            pltpu.VMEM((CROWS * H,), jnp.float32),
            pltpu.VMEM((CROWS * H,), jnp.float32),
            pltpu.SemaphoreType.DMA,
            pltpu.SemaphoreType.DMA,
        ],
    )(_sc_body)
    out_sc = sc(x_flat[:N_SC], word_ebd, pos_ebd)

    tc = pl.pallas_call(
        _tc_body,
        grid=(TC_BLOCKS,),
        in_specs=[
            pl.BlockSpec((TCR,), lambda i: (i,)),
            pl.BlockSpec((V, H), lambda i: (0, 0)),
            pl.BlockSpec((L, H), lambda i: (0, 0)),
        ],
        out_specs=pl.BlockSpec((TCR, H), lambda i: (i, 0)),
        out_shape=jax.ShapeDtypeStruct((N_TC, H), jnp.float32),
    )
    out_tc = tc(x_flat[N_SC:], word_ebd, pos_ebd)

    return jnp.concatenate([out_sc.reshape(N_SC, H), out_tc], axis=0)


def kernel(X, word_ebd, pos_ebd):
    out = _ebd(X.reshape(-1).astype(jnp.int32), word_ebd, pos_ebd)
    return out.reshape(B, L, H)


# speedup vs baseline: 1.1877x; 1.1877x over previous
import functools

import jax
import jax.numpy as jnp
from jax import lax
from jax.experimental import pallas as pl
from jax.experimental.pallas import tpu as pltpu
from jax.experimental.pallas import tpu_sc as plsc

B = 16384
L = 12
V = 29
H = 256
N = B * L
TROWS = L * V

N_SC = 36864
N_TC = N - N_SC
NW = 32
ROWS_PER_W = N_SC // NW
CROWS = 32
NCH = ROWS_PER_W // CROWS
LPAT = 384
TCR = 6144
TC_BLOCKS = N_TC // TCR


def _sc_body(x_hbm, word_hbm, pos_hbm, out_hbm,
             xv, lpat, wordv, posv, tab, bufa, bufb, sema, semb):
    c = lax.axis_index("c")
    s = lax.axis_index("s")
    wid = s * 2 + c
    base = wid * ROWS_PER_W
    iota = lax.iota(jnp.int32, 16)

    pltpu.sync_copy(x_hbm.at[pl.ds(base, ROWS_PER_W)], xv)
    pltpu.sync_copy(word_hbm, wordv)
    pltpu.sync_copy(pos_hbm, posv)

    def build_l(l, carry):
        pv = [posv[l, pl.ds(16 * j, 16)] for j in range(H // 16)]
        rb = (l * V) * H
        for v in range(V):
            for j in range(H // 16):
                tab[pl.ds(rb + v * H + 16 * j, 16)] = (
                    wordv[v, pl.ds(16 * j, 16)] + pv[j]
                )
        return carry

    lax.fori_loop(0, L, build_l, 0)

    for i in range(LPAT // 16):
        r = iota + jnp.int32(16 * i)
        lpat[pl.ds(16 * i, 16)] = lax.rem(r, jnp.int32(L)) * jnp.int32(V)

    def cstep(i, carry):
        ph = lax.rem(i, jnp.int32(LPAT // 16)) * 16
        xv[pl.ds(i * 16, 16)] = xv[pl.ds(i * 16, 16)] + lpat[pl.ds(ph, 16)]
        return carry

    lax.fori_loop(0, ROWS_PER_W // 16, cstep, 0)

    def fill(k, buf):
        def group(g, carry):
            rv = xv[pl.ds(k * CROWS + g * 16, 16)] * jnp.int32(H)
            for j in range(16):
                src = rv[j]
                dst = (g * 16 + j) * H
                vals = [tab[pl.ds(src + 16 * ci, 16)] for ci in range(H // 16)]
                for ci in range(H // 16):
                    buf[pl.ds(dst + 16 * ci, 16)] = vals[ci]
            return carry

        lax.fori_loop(0, CROWS // 16, group, 0)

    def start_write(k, buf, sem):
        pltpu.async_copy(
            buf, out_hbm.at[pl.ds((base + k * CROWS) * H, CROWS * H)], sem
        )

    def wait_write(buf, sem):
        pltpu.make_async_copy(
            buf, out_hbm.at[pl.ds(base * H, CROWS * H)], sem
        ).wait()

    fill(0, bufa)
    start_write(0, bufa, sema)
    fill(1, bufb)
    start_write(1, bufb, semb)

    def step(i, carry):
        k0 = 2 * i
        wait_write(bufa, sema)
        fill(k0, bufa)
        start_write(k0, bufa, sema)
        wait_write(bufb, semb)
        fill(k0 + 1, bufb)
        start_write(k0 + 1, bufb, semb)
        return carry

    lax.fori_loop(1, NCH // 2, step, 0)
    wait_write(bufa, sema)
    wait_write(bufb, semb)


def _tc_body(x_ref, word_ref, pos_ref, full_ref, o_ref):
    del full_ref
    word = word_ref[...]
    w1 = word.astype(jnp.bfloat16)
    r1 = word - w1.astype(jnp.float32)
    w2 = r1.astype(jnp.bfloat16)
    w3 = (r1 - w2.astype(jnp.float32)).astype(jnp.bfloat16)
    idx = x_ref[...].reshape(TCR, 1)
    onehot = (idx == lax.broadcasted_iota(jnp.int32, (TCR, V), 1)
              ).astype(jnp.bfloat16)

    def mm(w):
        return jnp.dot(onehot, w, preferred_element_type=jnp.float32)

    gathered = (mm(w1) + mm(w2)) + mm(w3)
    posrep = jnp.broadcast_to(
        pos_ref[...][None, :, :], (TCR // L, L, H)
    ).reshape(TCR, H)
    o_ref[...] = gathered + posrep


@jax.jit
def _ebd(x_flat, word_ebd, pos_ebd):
    mesh = plsc.VectorSubcoreMesh(core_axis_name="c", subcore_axis_name="s")
    sc = functools.partial(
        pl.kernel,
        mesh=mesh,
        compiler_params=pltpu.CompilerParams(
            needs_layout_passes=False,
            use_tc_tiling_on_sc=False,
        ),
        out_type=jax.ShapeDtypeStruct((N * H,), jnp.float32),
        scratch_types=[
            pltpu.VMEM((ROWS_PER_W,), jnp.int32),
            pltpu.VMEM((LPAT,), jnp.int32),
            pltpu.VMEM((V, H), jnp.float32),
            pltpu.VMEM((L, H), jnp.float32),
            pltpu.VMEM((TROWS * H,), jnp.float32),
            pltpu.VMEM((CROWS * H,), jnp.float32),
            pltpu.VMEM((CROWS * H,), jnp.float32),
            pltpu.SemaphoreType.DMA,
            pltpu.SemaphoreType.DMA,
        ],
    )(_sc_body)
    out_sc = sc(x_flat[:N_SC], word_ebd, pos_ebd)

    tc = pl.pallas_call(
        _tc_body,
        grid=(TC_BLOCKS,),
        in_specs=[
            pl.BlockSpec((TCR,), lambda i: (i,)),
            pl.BlockSpec((V, H), lambda i: (0, 0)),
            pl.BlockSpec((L, H), lambda i: (0, 0)),
            pl.BlockSpec(memory_space=pl.ANY),
        ],
        out_specs=pl.BlockSpec((TCR, H), lambda i: (i + N_SC // TCR, 0)),
        out_shape=jax.ShapeDtypeStruct((N, H), jnp.float32),
        input_output_aliases={3: 0},
    )
    return tc(x_flat[N_SC:], word_ebd, pos_ebd, out_sc.reshape(N, H))


def kernel(X, word_ebd, pos_ebd):
    out = _ebd(X.reshape(-1).astype(jnp.int32), word_ebd, pos_ebd)
    return out.reshape(B, L, H)
